# baseline (speedup 1.0000x reference)
import functools

import jax
import jax.numpy as jnp
from jax import lax
from jax.experimental import pallas as pl
from jax.experimental.pallas import tpu as pltpu
from jax.experimental.pallas import tpu_sc as plsc

BATCH = 256
N_CODES = 64
BOOK = 1024
D = 64
ROWS = BATCH * N_CODES

G = 8
BB = 32
_CH = 128

_DIST_PREC = jax.lax.Precision.DEFAULT


def _argmin_body(wq_ref, cb_ref, idx_ref, tab_ref):
    j = pl.program_id(0)
    iota2 = jax.lax.broadcasted_iota(jnp.int32, (BATCH, BOOK), 1)
    idx_cols = []
    for g in range(G):
        x = wq_ref[:, g * D:(g + 1) * D]
        cb = cb_ref[g]
        xc = jax.lax.dot_general(
            x, cb, (((1,), (0,)), ((), ())),
            precision=_DIST_PREC, preferred_element_type=jnp.float32)
        x_sq = jnp.sum(x * x, axis=1, keepdims=True)
        c_sq = jnp.sum(cb * cb, axis=0)[None, :]
        dist = x_sq + c_sq - 2.0 * xc
        m = jnp.min(dist, axis=1, keepdims=True)
        idx_cols.append(
            jnp.min(jnp.where(dist == m, iota2, BOOK), axis=1, keepdims=True))
        eye = (jax.lax.broadcasted_iota(jnp.int32, (D, D), 0) ==
               jax.lax.broadcasted_iota(jnp.int32, (D, D), 1)).astype(jnp.float32)
        tab_ref[g * BOOK:(g + 1) * BOOK, 0:D] = jax.lax.dot_general(
            cb, eye, (((0,), (0,)), ((), ())),
            precision=jax.lax.Precision.HIGHEST,
            preferred_element_type=jnp.float32)
    idxg = jnp.concatenate(idx_cols, axis=1)
    for jj in range(N_CODES // G):
        @pl.when(j == jj)
        def _():
            idx_ref[:, jj * G:(jj + 1) * G] = idxg


def _argmin_call(w_q, codebook):
    cb_t = jnp.transpose(codebook, (0, 2, 1))
    return pl.pallas_call(
        _argmin_body,
        grid=(N_CODES // G,),
        in_specs=[
            pl.BlockSpec((BATCH, G * D), lambda j: (0, j)),
            pl.BlockSpec((G, D, BOOK), lambda j: (j, 0, 0)),
        ],
        out_specs=[
            pl.BlockSpec((BATCH, N_CODES), lambda j: (0, 0)),
            pl.BlockSpec((G * BOOK, 2 * D), lambda j: (j, 0)),
        ],
        out_shape=[
            jax.ShapeDtypeStruct((BATCH, N_CODES), jnp.int32),
            jax.ShapeDtypeStruct((N_CODES * BOOK, 2 * D), jnp.float32),
        ],
    )(w_q, cb_t)


def _onehot_body(idx_ref, oh_ref):
    idx2 = idx_ref[...]
    iota3 = jax.lax.broadcasted_iota(jnp.int32, (BB, N_CODES, BOOK), 2)
    oh_ref[...] = (idx2[:, :, None] == iota3).astype(jnp.float32)


def _onehot_call(idx):
    return pl.pallas_call(
        _onehot_body,
        grid=(BATCH // BB,),
        in_specs=[pl.BlockSpec((BB, N_CODES), lambda i: (i, 0))],
        out_specs=pl.BlockSpec((BB, N_CODES, BOOK), lambda i: (i, 0, 0)),
        out_shape=jax.ShapeDtypeStruct((BATCH, N_CODES, BOOK), jnp.float32),
    )(idx)


def _sc_gather(table, w_q, idx):
    info = plsc.get_sparse_core_info()
    nw = info.num_cores * info.num_subcores
    rpw = ROWS // nw
    n_chunks = rpw // _CH
    rows_per_chunk = _CH // N_CODES
    mesh = plsc.VectorSubcoreMesh(core_axis_name="c", subcore_axis_name="s")

    @functools.partial(
        pl.kernel, mesh=mesh,
        out_type=[
            jax.ShapeDtypeStruct((BATCH, N_CODES * D), jnp.float32),
            jax.ShapeDtypeStruct((BATCH, N_CODES * D), jnp.float32),
        ],
        scratch_types=[
            pltpu.VMEM((rows_per_chunk, N_CODES), jnp.int32),
            pltpu.VMEM((_CH,), jnp.int32),
            pltpu.VMEM((_CH, 2 * D), jnp.float32),
            pltpu.VMEM((rows_per_chunk, N_CODES * D), jnp.float32),
            pltpu.VMEM((rows_per_chunk, N_CODES * D), jnp.float32),
            pltpu.SemaphoreType.DMA,
        ],
    )
    def k(table_hbm, wq_hbm, idx_hbm, w_hbm, we_hbm,
          idx_v, gidx_v, rows_v, wq_v, we_v, sem):
        wid = lax.axis_index("s") * info.num_cores + lax.axis_index("c")
        lane_iota = lax.iota(jnp.int32, 16)
        base_row = wid * (rpw // N_CODES)

        def chunk_body(c, _):
            crow = base_row + c * rows_per_chunk
            pltpu.sync_copy(idx_hbm.at[pl.ds(crow, rows_per_chunk)], idx_v)
            for v in range(_CH // 16):
                row = v * 16 // N_CODES
                sl = pl.ds((v * 16) % N_CODES, 16)
                slot16 = (v * 16) % N_CODES + lane_iota
                gidx_v[pl.ds(v * 16, 16)] = idx_v[row, sl] + slot16 * BOOK
            gather = pltpu.async_copy(table_hbm.at[gidx_v], rows_v, sem)
            pltpu.sync_copy(wq_hbm.at[pl.ds(crow, rows_per_chunk)], wq_v)
            gather.wait()

            def grp(g, _):
                row_g = g * 16 // N_CODES
                for lane in range(16):
                    r = g * 16 + lane
                    col = ((g * 16) % N_CODES + lane) * D
                    for d4 in range(D // 16):
                        val = rows_v[r, pl.ds(d4 * 16, 16)]
                        csl = pl.ds(col + d4 * 16, 16)
                        we_v[row_g, csl] = val
                        q = wq_v[row_g, csl]
                        wq_v[row_g, csl] = q + (val - q)
                return 0

            lax.fori_loop(0, _CH // 16, grp, 0)
            pltpu.sync_copy(we_v, we_hbm.at[pl.ds(crow, rows_per_chunk)])
            pltpu.sync_copy(wq_v, w_hbm.at[pl.ds(crow, rows_per_chunk)])
            return 0

        lax.fori_loop(0, n_chunks, chunk_body, 0)

    return k(table, w_q, idx)


def kernel(w_q, codebook):
    idx, table = _argmin_call(w_q, codebook)
    one_hot = _onehot_call(idx)
    w, w_e = _sc_gather(table, w_q, idx)
    return w, w_e, one_hot


# speedup vs baseline: 1.0889x; 1.0889x over previous
import functools

import jax
import jax.numpy as jnp
from jax import lax
from jax.experimental import pallas as pl
from jax.experimental.pallas import tpu as pltpu
from jax.experimental.pallas import tpu_sc as plsc

BATCH = 256
N_CODES = 64
BOOK = 1024
D = 64
ROWS = BATCH * N_CODES

G = 8
BB = 32
_CH = 128

_DIST_PREC = jax.lax.Precision.DEFAULT


def _argmin_body(wq_ref, cb_ref, idx_ref, tab_ref):
    j = pl.program_id(0)
    iota2 = jax.lax.broadcasted_iota(jnp.int32, (BATCH, BOOK), 1)
    idx_cols = []
    for g in range(G):
        x = wq_ref[:, g * D:(g + 1) * D]
        cb = cb_ref[g]
        xc = jax.lax.dot_general(
            x, cb, (((1,), (0,)), ((), ())),
            precision=_DIST_PREC, preferred_element_type=jnp.float32)
        x_sq = jnp.sum(x * x, axis=1, keepdims=True)
        c_sq = jnp.sum(cb * cb, axis=0)[None, :]
        dist = x_sq + c_sq - 2.0 * xc
        m = jnp.min(dist, axis=1, keepdims=True)
        idx_cols.append(
            jnp.min(jnp.where(dist == m, iota2, BOOK), axis=1, keepdims=True))
        tab_ref[g * BOOK:(g + 1) * BOOK, 0:D] = jnp.transpose(cb, (1, 0))
    idxg = jnp.concatenate(idx_cols, axis=1)
    for jj in range(N_CODES // G):
        @pl.when(j == jj)
        def _():
            idx_ref[:, jj * G:(jj + 1) * G] = idxg


def _argmin_call(w_q, codebook):
    cb_t = jnp.transpose(codebook, (0, 2, 1))
    return pl.pallas_call(
        _argmin_body,
        grid=(N_CODES // G,),
        in_specs=[
            pl.BlockSpec((BATCH, G * D), lambda j: (0, j)),
            pl.BlockSpec((G, D, BOOK), lambda j: (j, 0, 0)),
        ],
        out_specs=[
            pl.BlockSpec((BATCH, N_CODES), lambda j: (0, 0)),
            pl.BlockSpec((G * BOOK, 2 * D), lambda j: (j, 0)),
        ],
        out_shape=[
            jax.ShapeDtypeStruct((BATCH, N_CODES), jnp.int32),
            jax.ShapeDtypeStruct((N_CODES * BOOK, 2 * D), jnp.float32),
        ],
    )(w_q, cb_t)


def _onehot_body(idx_ref, oh_ref):
    idx2 = idx_ref[...]
    iota3 = jax.lax.broadcasted_iota(jnp.int32, (BB, N_CODES, BOOK), 2)
    oh_ref[...] = (idx2[:, :, None] == iota3).astype(jnp.float32)


def _onehot_call(idx):
    return pl.pallas_call(
        _onehot_body,
        grid=(BATCH // BB,),
        in_specs=[pl.BlockSpec((BB, N_CODES), lambda i: (i, 0))],
        out_specs=pl.BlockSpec((BB, N_CODES, BOOK), lambda i: (i, 0, 0)),
        out_shape=jax.ShapeDtypeStruct((BATCH, N_CODES, BOOK), jnp.float32),
    )(idx)


def _sc_gather(table, w_q, idx):
    info = plsc.get_sparse_core_info()
    nw = info.num_cores * info.num_subcores
    rpw = ROWS // nw
    n_chunks = rpw // _CH
    rows_per_chunk = _CH // N_CODES
    mesh = plsc.VectorSubcoreMesh(core_axis_name="c", subcore_axis_name="s")

    @functools.partial(
        pl.kernel, mesh=mesh,
        out_type=[
            jax.ShapeDtypeStruct((BATCH, N_CODES * D), jnp.float32),
            jax.ShapeDtypeStruct((BATCH, N_CODES * D), jnp.float32),
        ],
        scratch_types=[
            pltpu.VMEM((rows_per_chunk, N_CODES), jnp.int32),
            pltpu.VMEM((_CH,), jnp.int32),
            pltpu.VMEM((_CH, 2 * D), jnp.float32),
            pltpu.VMEM((rows_per_chunk, N_CODES * D), jnp.float32),
            pltpu.VMEM((rows_per_chunk, N_CODES * D), jnp.float32),
            pltpu.SemaphoreType.DMA,
        ],
    )
    def k(table_hbm, wq_hbm, idx_hbm, w_hbm, we_hbm,
          idx_v, gidx_v, rows_v, wq_v, we_v, sem):
        wid = lax.axis_index("s") * info.num_cores + lax.axis_index("c")
        lane_iota = lax.iota(jnp.int32, 16)
        base_row = wid * (rpw // N_CODES)

        def chunk_body(c, _):
            crow = base_row + c * rows_per_chunk
            pltpu.sync_copy(idx_hbm.at[pl.ds(crow, rows_per_chunk)], idx_v)
            for v in range(_CH // 16):
                row = v * 16 // N_CODES
                sl = pl.ds((v * 16) % N_CODES, 16)
                slot16 = (v * 16) % N_CODES + lane_iota
                gidx_v[pl.ds(v * 16, 16)] = idx_v[row, sl] + slot16 * BOOK
            gather = pltpu.async_copy(table_hbm.at[gidx_v], rows_v, sem)
            pltpu.sync_copy(wq_hbm.at[pl.ds(crow, rows_per_chunk)], wq_v)
            gather.wait()

            def grp(g, _):
                row_g = g * 16 // N_CODES
                for lane in range(16):
                    r = g * 16 + lane
                    col = ((g * 16) % N_CODES + lane) * D
                    for d4 in range(D // 16):
                        val = rows_v[r, pl.ds(d4 * 16, 16)]
                        csl = pl.ds(col + d4 * 16, 16)
                        we_v[row_g, csl] = val
                        q = wq_v[row_g, csl]
                        wq_v[row_g, csl] = q + (val - q)
                return 0

            lax.fori_loop(0, _CH // 16, grp, 0)
            pltpu.sync_copy(we_v, we_hbm.at[pl.ds(crow, rows_per_chunk)])
            pltpu.sync_copy(wq_v, w_hbm.at[pl.ds(crow, rows_per_chunk)])
            return 0

        lax.fori_loop(0, n_chunks, chunk_body, 0)

    return k(table, w_q, idx)


def kernel(w_q, codebook):
    idx, table = _argmin_call(w_q, codebook)
    one_hot = _onehot_call(idx)
    w, w_e = _sc_gather(table, w_q, idx)
    return w, w_e, one_hot
